# baseline (speedup 1.0000x reference)
import functools
import math

import jax
import jax.numpy as jnp
from jax import lax
from jax.experimental import pallas as pl
from jax.experimental.pallas import tpu as pltpu
from jax.experimental.pallas import tpu_sc as plsc

NC = 2
NS = 16
NW = NC * NS
LANES = 16
NBUF = 6
PRE = 3


@jax.jit
def _sc_embed(lut, xt):
    hist, batch = xt.shape
    _, d = lut.shape
    bcols = batch // NW
    n = hist
    n_groups = (n + NBUF - 1) // NBUF
    scale = float(math.sqrt(d))
    mesh = plsc.VectorSubcoreMesh(
        core_axis_name="c", subcore_axis_name="s", num_cores=NC, num_subcores=NS
    )

    @functools.partial(
        pl.kernel,
        out_type=jax.ShapeDtypeStruct((hist, batch, d), jnp.float32),
        mesh=mesh,
        scratch_types=[
            pltpu.VMEM((hist, bcols), jnp.int32),
            pltpu.VMEM((NBUF, bcols, d), jnp.float32),
        ]
        + [pltpu.SemaphoreType.DMA] * (2 * NBUF),
    )
    def k(lut_hbm, idx_hbm, out_hbm, idx_v, bufs, *sems):
        gsem = sems[:NBUF]
        ssem = sems[NBUF:]
        wid = lax.axis_index("s") * NC + lax.axis_index("c")
        base = wid * bcols
        pltpu.sync_copy(idx_hbm.at[:, pl.ds(base, bcols)], idx_v)

        def gather(j, b):
            return pltpu.make_async_copy(
                lut_hbm.at[idx_v.at[j]], bufs.at[b], gsem[b]
            )

        def scatter(j, b):
            return pltpu.make_async_copy(
                bufs.at[b], out_hbm.at[j, pl.ds(base, bcols)], ssem[b]
            )

        for b in range(PRE):
            gather(b, b).start()

        def group(g, carry):
            for b in range(NBUF):
                j = g * NBUF + b
                jp = j + PRE
                bp = (b + PRE) % NBUF

                @pl.when((jp < n) & (j >= NBUF - PRE))
                def _():
                    scatter(jp - NBUF, bp).wait()
                    gather(jp, bp).start()

                @pl.when((jp < n) & (j < NBUF - PRE))
                def _():
                    gather(jp, bp).start()

                @pl.when(j < n)
                def _():
                    gather(j, b).wait()

                    @plsc.parallel_loop(0, bcols, step=1, unroll=2)
                    def _(i):
                        for t in range(d // LANES):
                            sl = pl.ds(t * LANES, LANES)
                            bufs[b, i, sl] = bufs[b, i, sl] * scale

                    scatter(j, b).start()
            return carry

        lax.fori_loop(0, n_groups, group, 0)

        for j in range(max(0, n - NBUF), n):
            scatter(j, j % NBUF).wait()

    return k(lut, xt)


def kernel(x, lut):
    out_t = _sc_embed(lut, x.T)
    return out_t.transpose(1, 0, 2)
